# baseline (speedup 1.0000x reference)
import functools

import jax
import jax.numpy as jnp
from jax import lax
from jax.experimental import pallas as pl
from jax.experimental.pallas import tpu as pltpu
from jax.experimental.pallas import tpu_sc as plsc

D_MODEL = 64
MAX_LEN = 200
BATCH = 4096
NUM_WORKERS = 32
NPAIR = BATCH // 2
PPW = NPAIR // NUM_WORKERS
LANES = 16
NBUF = 3
HALF = MAX_LEN // 2
ROWS = 2 * HALF
OUT_ROWS = NPAIR * ROWS
G1 = 104
G2 = ROWS - G1


def _pos_encoding():
    even_i = jnp.arange(0, D_MODEL, 2).astype(jnp.float32)
    denominator = jnp.power(10000.0, even_i / D_MODEL)
    position = jnp.arange(MAX_LEN, dtype=jnp.float32).reshape(MAX_LEN, 1)
    even_pe = jnp.sin(position / denominator)
    odd_pe = jnp.cos(position / denominator)
    return jnp.stack([even_pe, odd_pe], axis=2).reshape(MAX_LEN, D_MODEL)


def kernel(indices, table):
    pe = _pos_encoding()
    pe_eo = jnp.concatenate(
        [jnp.tile(pe[0::2], (2, 1)), jnp.tile(pe[1::2], (2, 1))], axis=0)
    ind3 = indices.reshape(NPAIR, 2, MAX_LEN)
    idx_eo = jnp.concatenate(
        [ind3[:, 0, 0::2], ind3[:, 1, 0::2], ind3[:, 0, 1::2], ind3[:, 1, 1::2]],
        axis=1)

    mesh = plsc.VectorSubcoreMesh(core_axis_name="c", subcore_axis_name="s")

    @functools.partial(
        pl.kernel,
        mesh=mesh,
        compiler_params=pltpu.CompilerParams(use_tc_tiling_on_sc=False),
        out_type=jax.ShapeDtypeStruct((OUT_ROWS, 2 * D_MODEL), jnp.float32),
        scratch_types=[
            pltpu.VMEM((PPW, 2 * ROWS), jnp.int32),
            pltpu.VMEM((2 * ROWS, D_MODEL), jnp.float32),
            pltpu.VMEM((NBUF, 2, ROWS, D_MODEL), jnp.float32),
            pltpu.SemaphoreType.DMA((NBUF,)),
            pltpu.SemaphoreType.DMA((NBUF,)),
        ],
    )
    def k(idx_hbm, table_hbm, pe_hbm, out_hbm, idx_v, pe_v, bufs, gsem, osem):
        wid = lax.axis_index("s") * 2 + lax.axis_index("c")
        p0 = wid * PPW
        pltpu.sync_copy(idx_hbm.at[pl.ds(p0, PPW)], idx_v)
        pltpu.sync_copy(pe_hbm, pe_v)

        def start_gathers(i, s):
            for h in range(2):
                io = h * ROWS
                pltpu.async_copy(
                    table_hbm.at[idx_v.at[i, pl.ds(io, G1)]],
                    bufs.at[s, h, pl.ds(0, G1)], gsem.at[s])
                pltpu.async_copy(
                    table_hbm.at[idx_v.at[i, pl.ds(io + G1, G2)]],
                    bufs.at[s, h, pl.ds(G1, G2)], gsem.at[s])

        def wait_gathers(s):
            for h in range(2):
                pltpu.make_async_copy(
                    out_hbm.at[pl.ds(0, ROWS), pl.ds(0, D_MODEL)],
                    bufs.at[s, h], gsem.at[s]).wait()

        def wait_out(s):
            for h in range(2):
                pltpu.make_async_copy(
                    bufs.at[s, h],
                    out_hbm.at[pl.ds(0, ROWS), pl.ds(0, D_MODEL)],
                    osem.at[s]).wait()

        def process(i, s):
            wait_gathers(s)

            def row(r, c):
                for h in range(2):
                    for j in range(D_MODEL // LANES):
                        plsc.addupdate(
                            bufs.at[s, h, r, pl.ds(LANES * j, LANES)],
                            pe_v[h * ROWS + r, pl.ds(LANES * j, LANES)])
                return c

            lax.fori_loop(0, ROWS, row, 0)
            base = (p0 + i) * ROWS
            for h in range(2):
                pltpu.async_copy(
                    bufs.at[s, h],
                    out_hbm.at[pl.ds(base, ROWS), pl.ds(h * D_MODEL, D_MODEL)],
                    osem.at[s])

        def outer(io, carry):
            for s in range(NBUF):
                i = io * NBUF + s

                @pl.when(io >= 1)
                def _():
                    wait_out(s)

                start_gathers(i, s)
                if s == 0:
                    @pl.when(io >= 1)
                    def _():
                        process(io * NBUF - 1, NBUF - 1)
                else:
                    process(i - 1, s - 1)
            return carry

        lax.fori_loop(0, PPW // NBUF, outer, 0)
        for i in range((PPW // NBUF) * NBUF, PPW):
            s = i % NBUF
            wait_out(s)
            start_gathers(i, s)
            process(i - 1, (i - 1) % NBUF)
        process(PPW - 1, (PPW - 1) % NBUF)
        for s in range(NBUF):
            wait_out(s)

    out2d = k(idx_eo, table, pe_eo)
    return out2d.reshape(BATCH, MAX_LEN, D_MODEL)
